# Initial kernel scaffold; baseline (speedup 1.0000x reference)
import jax
import jax.numpy as jnp
from jax.experimental import pallas as pl


def kernel(x, edge_index, batch, W1_0, B1_0, W2_0, B2_0, G_0, BT_0, W1_1, B1_1, W2_1, B2_1, G_1, BT_1, W1_2, B1_2, W2_2, B2_2, G_2, BT_2):
    raise NotImplementedError("write your pallas kernel here")


# speedup vs baseline: 4.5318x; 4.5318x over previous
import functools

import jax
import jax.numpy as jnp
from jax import lax
from jax.experimental import pallas as pl
from jax.experimental.pallas import tpu as pltpu
from jax.experimental.pallas import tpu_sc as plsc

N, D, E, G = 10000, 128, 320000, 128


NC, NS = 2, 16
NW = NC * NS
EPW = E // NW
CHUNK = 80
NCHUNK = EPW // CHUNK
WBT = 10
RPT = N // WBT
ZROWS = 200

def _sc_agg_body(z_hbm, src_hbm, dst_hbm, out_hbm, acc_sh, src_v, dst_v,
                 rows_v, zero_v, sem):
    cid = lax.axis_index("c")
    sid = lax.axis_index("s")
    wid = cid * NS + sid

    @pl.when(sid < WBT)
    def _zero():
        def _zrow(i, carry):
            for j in range(D // 16):
                zero_v[i, pl.ds(j * 16, 16)] = jnp.zeros((16,), jnp.float32)
            return carry

        lax.fori_loop(0, ZROWS, _zrow, 0)
        for k in range(RPT // ZROWS):
            pltpu.sync_copy(
                zero_v, acc_sh.at[pl.ds(sid * RPT + k * ZROWS, ZROWS)])

    plsc.subcore_barrier()

    base = wid * EPW

    def _edges(i, carry):
        off = pl.multiple_of(base + i * CHUNK, 8)
        pltpu.sync_copy(src_hbm.at[pl.ds(off, CHUNK)], src_v)
        pltpu.sync_copy(dst_hbm.at[pl.ds(off, CHUNK)], dst_v)
        pltpu.async_copy(z_hbm.at[src_v], rows_v, sem).wait()
        pltpu.sync_copy(rows_v, acc_sh.at[dst_v], add=True)
        return carry

    lax.fori_loop(0, NCHUNK, _edges, 0)
    plsc.subcore_barrier()

    @pl.when(sid < WBT)
    def _writeback():
        pltpu.sync_copy(acc_sh.at[pl.ds(sid * RPT, RPT)],
                        out_hbm.at[cid, pl.ds(sid * RPT, RPT)])


@functools.lru_cache(maxsize=None)
def _get_sc_agg():
    mesh = plsc.VectorSubcoreMesh(core_axis_name="c", subcore_axis_name="s")
    return pl.kernel(
        _sc_agg_body,
        out_type=jax.ShapeDtypeStruct((NC, N, D), jnp.float32),
        mesh=mesh,
        scratch_types=[
            pltpu.VMEM_SHARED((N, D), jnp.float32),
            pltpu.VMEM((CHUNK,), jnp.int32),
            pltpu.VMEM((CHUNK,), jnp.int32),
            pltpu.VMEM((CHUNK, D), jnp.float32),
            pltpu.VMEM((ZROWS, D), jnp.float32),
            pltpu.SemaphoreType.DMA,
        ],
    )


BLK = 2000
NBLK = N // BLK


def _mlp_body(z_ref, p0_ref, p1_ref, w1_ref, b1_ref, w2_ref, b2_ref,
              c_ref, s_ref):
    h = z_ref[...] + p0_ref[...] + p1_ref[...]
    a = jnp.maximum(
        jnp.dot(h, w1_ref[...],
                preferred_element_type=jnp.float32) + b1_ref[...], 0.0)
    b = jnp.dot(a, w2_ref[...],
                preferred_element_type=jnp.float32) + b2_ref[...]
    c = jnp.maximum(b, 0.0)
    c_ref[...] = c

    bsum = jnp.sum(c, axis=0, keepdims=True)
    bmean = bsum * (1.0 / BLK)
    bm2 = jnp.sum((c - bmean) ** 2, axis=0, keepdims=True)
    s_ref[...] = jnp.concatenate(
        [bsum, bm2, jnp.zeros((6, D), jnp.float32)], axis=0)[None]


def _mlp_call(z, p0, p1, W1, B1, W2, B2):
    return pl.pallas_call(
        _mlp_body,
        grid=(NBLK,),
        in_specs=[
            pl.BlockSpec((BLK, D), lambda i: (i, 0)),
            pl.BlockSpec((BLK, D), lambda i: (i, 0)),
            pl.BlockSpec((BLK, D), lambda i: (i, 0)),
            pl.BlockSpec((D, D), lambda i: (0, 0)),
            pl.BlockSpec((1, D), lambda i: (0, 0)),
            pl.BlockSpec((D, D), lambda i: (0, 0)),
            pl.BlockSpec((1, D), lambda i: (0, 0)),
        ],
        out_specs=[
            pl.BlockSpec((BLK, D), lambda i: (i, 0)),
            pl.BlockSpec((1, 8, D), lambda i: (i, 0, 0)),
        ],
        out_shape=[
            jax.ShapeDtypeStruct((N, D), jnp.float32),
            jax.ShapeDtypeStruct((NBLK, 8, D), jnp.float32),
        ],
    )(z, p0, p1, W1, B1.reshape(1, D), W2, B2.reshape(1, D))


def _norm_body(c_ref, s_ref, g_ref, bt_ref, batch_ref, z_ref, pool_ref):
    s = s_ref[...]
    bsum = s[:, 0, :]
    bm2 = s[:, 1, :]
    mean = jnp.sum(bsum, axis=0, keepdims=True) * (1.0 / N)
    dev = bsum * (1.0 / BLK) - mean
    var = (jnp.sum(bm2, axis=0, keepdims=True)
           + BLK * jnp.sum(dev * dev, axis=0, keepdims=True)) * (1.0 / N)
    inv = lax.rsqrt(var + 1e-5) * g_ref[...]
    zb = (c_ref[...] - mean) * inv + bt_ref[...]
    z_ref[...] = zb

    brow = batch_ref[0, 0, :]
    onehot = (brow[None, :] ==
              lax.broadcasted_iota(jnp.int32, (G, BLK), 0)).astype(jnp.float32)
    contrib = jnp.dot(onehot, zb, preferred_element_type=jnp.float32,
                      precision=lax.Precision.HIGHEST)

    @pl.when(pl.program_id(0) == 0)
    def _():
        pool_ref[...] = jnp.zeros_like(pool_ref)

    pool_ref[...] += contrib


def _norm_call(c, sums, Gm, BT, batch3):
    return pl.pallas_call(
        _norm_body,
        grid=(NBLK,),
        in_specs=[
            pl.BlockSpec((BLK, D), lambda i: (i, 0)),
            pl.BlockSpec((NBLK, 8, D), lambda i: (0, 0, 0)),
            pl.BlockSpec((1, D), lambda i: (0, 0)),
            pl.BlockSpec((1, D), lambda i: (0, 0)),
            pl.BlockSpec((1, 1, BLK), lambda i: (i, 0, 0)),
        ],
        out_specs=[
            pl.BlockSpec((BLK, D), lambda i: (i, 0)),
            pl.BlockSpec((G, D), lambda i: (0, 0)),
        ],
        out_shape=[
            jax.ShapeDtypeStruct((N, D), jnp.float32),
            jax.ShapeDtypeStruct((G, D), jnp.float32),
        ],
    )(c, sums, Gm.reshape(1, D), BT.reshape(1, D), batch3)


def kernel(x, edge_index, batch, W1_0, B1_0, W2_0, B2_0, G_0, BT_0,
           W1_1, B1_1, W2_1, B2_1, G_1, BT_1,
           W1_2, B1_2, W2_2, B2_2, G_2, BT_2):
    src = edge_index[0]
    dst = edge_index[1]
    batch3 = batch.reshape(NBLK, 1, BLK)
    params = [(W1_0, B1_0, W2_0, B2_0, G_0, BT_0),
              (W1_1, B1_1, W2_1, B2_1, G_1, BT_1),
              (W1_2, B1_2, W2_2, B2_2, G_2, BT_2)]
    z = x
    pools = []
    sc_agg = _get_sc_agg()
    for (W1, B1, W2, B2, Gm, BT) in params:
        parts = sc_agg(z, src, dst)
        c, sums = _mlp_call(z, parts[0], parts[1], W1, B1, W2, B2)
        z, pool = _norm_call(c, sums, Gm, BT, batch3)
        pools.append(pool)
    return jnp.concatenate(pools, axis=1)


# speedup vs baseline: 8.0098x; 1.7674x over previous
import functools

import jax
import jax.numpy as jnp
from jax import lax
from jax.experimental import pallas as pl
from jax.experimental.pallas import tpu as pltpu
from jax.experimental.pallas import tpu_sc as plsc

N, D, E, G = 10000, 128, 320000, 128


NC, NS = 2, 16
NW = NC * NS
EPW = E // NW
CHUNK = 80
NCHUNK = EPW // CHUNK
NBUF = 2
VCH = NCHUNK + 1
WBT = 10
RPT = N // WBT
ZROWS = 40

def _sc_agg_body(z_hbm, src_hbm, dst_hbm, out_hbm, acc_sh, src_all, dst_all,
                 buf_0, buf_1, gs_0, gs_1, ss_0, ss_1):
    cid = lax.axis_index("c")
    sid = lax.axis_index("s")
    wid = cid * NS + sid

    bufs = (buf_0, buf_1)
    gsems = (gs_0, gs_1)
    ssems = (ss_0, ss_1)

    pltpu.sync_copy(src_hbm.at[wid], src_all)
    pltpu.sync_copy(dst_hbm.at[wid], dst_all)

    def _zrow(i, carry):
        for j in range(D // 16):
            buf_0[i, pl.ds(j * 16, 16)] = jnp.zeros((16,), jnp.float32)
        return carry

    lax.fori_loop(0, ZROWS, _zrow, 0)

    @pl.when(sid < WBT)
    def _zero():
        for k in range(RPT // ZROWS):
            pltpu.sync_copy(buf_0.at[pl.ds(0, ZROWS)],
                            acc_sh.at[pl.ds(sid * RPT + k * ZROWS, ZROWS)])

    plsc.subcore_barrier()

    def fire_gather(ci, b):
        off = pl.multiple_of(ci * CHUNK, 16)
        pltpu.async_copy(z_hbm.at[src_all.at[pl.ds(off, CHUNK)]], bufs[b],
                         gsems[b])

    def wait_gather(b):
        pltpu.make_async_copy(z_hbm.at[src_all.at[pl.ds(0, CHUNK)]], bufs[b],
                              gsems[b]).wait()

    def fire_scatter(ci, b):
        pltpu.async_copy(bufs[b], acc_sh.at[dst_all.at[ci]], ssems[b],
                         add=True)

    def drain_scatter(b):
        pltpu.make_async_copy(bufs[b], acc_sh.at[dst_all.at[0]],
                              ssems[b]).wait()

    PD = NBUF - 1
    for k in range(NBUF):
        fire_gather(k, k)

    def _step(j, carry):
        for k in range(NBUF):
            i = NBUF * j + k
            wait_gather(k)

            @pl.when(i < NCHUNK)
            def _(k=k, i=i):
                fire_scatter(i, k)

            nb = (k + PD) % NBUF

            @pl.when(jnp.logical_and(i >= 1, i <= NCHUNK))
            def _(nb=nb):
                drain_scatter(nb)

            @pl.when(jnp.logical_and(i >= 1, i + PD <= NCHUNK))
            def _(nb=nb, i=i):
                fire_gather(lax.rem(i + PD, NCHUNK), nb)

        return carry

    lax.fori_loop(0, VCH // NBUF, _step, 0)
    plsc.subcore_barrier()

    @pl.when(sid < WBT)
    def _writeback():
        pltpu.sync_copy(acc_sh.at[pl.ds(sid * RPT, RPT)],
                        out_hbm.at[cid, pl.ds(sid * RPT, RPT)])


@functools.lru_cache(maxsize=None)
def _get_sc_agg():
    mesh = plsc.VectorSubcoreMesh(core_axis_name="c", subcore_axis_name="s")
    return pl.kernel(
        _sc_agg_body,
        out_type=jax.ShapeDtypeStruct((NC, N, D), jnp.float32),
        mesh=mesh,
        scratch_types=[
            pltpu.VMEM_SHARED((N, D), jnp.float32),
            pltpu.VMEM((EPW,), jnp.int32),
            pltpu.VMEM((NCHUNK, CHUNK), jnp.int32),
            pltpu.VMEM((CHUNK, D), jnp.float32),
            pltpu.VMEM((CHUNK, D), jnp.float32),
            pltpu.SemaphoreType.DMA,
            pltpu.SemaphoreType.DMA,
            pltpu.SemaphoreType.DMA,
            pltpu.SemaphoreType.DMA,
        ],
    )


BLK = 2000
NBLK = N // BLK


def _mlp_body(z_ref, p0_ref, p1_ref, w1_ref, b1_ref, w2_ref, b2_ref,
              c_ref, s_ref):
    h = z_ref[...] + p0_ref[...] + p1_ref[...]
    a = jnp.maximum(
        jnp.dot(h, w1_ref[...],
                preferred_element_type=jnp.float32) + b1_ref[...], 0.0)
    b = jnp.dot(a, w2_ref[...],
                preferred_element_type=jnp.float32) + b2_ref[...]
    c = jnp.maximum(b, 0.0)
    c_ref[...] = c

    bsum = jnp.sum(c, axis=0, keepdims=True)
    bmean = bsum * (1.0 / BLK)
    bm2 = jnp.sum((c - bmean) ** 2, axis=0, keepdims=True)
    s_ref[...] = jnp.concatenate(
        [bsum, bm2, jnp.zeros((6, D), jnp.float32)], axis=0)[None]


def _mlp_call(z, p0, p1, W1, B1, W2, B2):
    return pl.pallas_call(
        _mlp_body,
        grid=(NBLK,),
        in_specs=[
            pl.BlockSpec((BLK, D), lambda i: (i, 0)),
            pl.BlockSpec((BLK, D), lambda i: (i, 0)),
            pl.BlockSpec((BLK, D), lambda i: (i, 0)),
            pl.BlockSpec((D, D), lambda i: (0, 0)),
            pl.BlockSpec((1, D), lambda i: (0, 0)),
            pl.BlockSpec((D, D), lambda i: (0, 0)),
            pl.BlockSpec((1, D), lambda i: (0, 0)),
        ],
        out_specs=[
            pl.BlockSpec((BLK, D), lambda i: (i, 0)),
            pl.BlockSpec((1, 8, D), lambda i: (i, 0, 0)),
        ],
        out_shape=[
            jax.ShapeDtypeStruct((N, D), jnp.float32),
            jax.ShapeDtypeStruct((NBLK, 8, D), jnp.float32),
        ],
    )(z, p0, p1, W1, B1.reshape(1, D), W2, B2.reshape(1, D))


def _norm_body(c_ref, s_ref, g_ref, bt_ref, batch_ref, z_ref, pool_ref):
    s = s_ref[...]
    bsum = s[:, 0, :]
    bm2 = s[:, 1, :]
    mean = jnp.sum(bsum, axis=0, keepdims=True) * (1.0 / N)
    dev = bsum * (1.0 / BLK) - mean
    var = (jnp.sum(bm2, axis=0, keepdims=True)
           + BLK * jnp.sum(dev * dev, axis=0, keepdims=True)) * (1.0 / N)
    inv = lax.rsqrt(var + 1e-5) * g_ref[...]
    zb = (c_ref[...] - mean) * inv + bt_ref[...]
    z_ref[...] = zb

    brow = batch_ref[0, 0, :]
    onehot = (brow[None, :] ==
              lax.broadcasted_iota(jnp.int32, (G, BLK), 0)).astype(jnp.float32)
    contrib = jnp.dot(onehot, zb, preferred_element_type=jnp.float32,
                      precision=lax.Precision.HIGHEST)

    @pl.when(pl.program_id(0) == 0)
    def _():
        pool_ref[...] = jnp.zeros_like(pool_ref)

    pool_ref[...] += contrib


def _norm_call(c, sums, Gm, BT, batch3):
    return pl.pallas_call(
        _norm_body,
        grid=(NBLK,),
        in_specs=[
            pl.BlockSpec((BLK, D), lambda i: (i, 0)),
            pl.BlockSpec((NBLK, 8, D), lambda i: (0, 0, 0)),
            pl.BlockSpec((1, D), lambda i: (0, 0)),
            pl.BlockSpec((1, D), lambda i: (0, 0)),
            pl.BlockSpec((1, 1, BLK), lambda i: (i, 0, 0)),
        ],
        out_specs=[
            pl.BlockSpec((BLK, D), lambda i: (i, 0)),
            pl.BlockSpec((G, D), lambda i: (0, 0)),
        ],
        out_shape=[
            jax.ShapeDtypeStruct((N, D), jnp.float32),
            jax.ShapeDtypeStruct((G, D), jnp.float32),
        ],
    )(c, sums, Gm.reshape(1, D), BT.reshape(1, D), batch3)


def kernel(x, edge_index, batch, W1_0, B1_0, W2_0, B2_0, G_0, BT_0,
           W1_1, B1_1, W2_1, B2_1, G_1, BT_1,
           W1_2, B1_2, W2_2, B2_2, G_2, BT_2):
    src = edge_index[0].reshape(NW, EPW)
    dst = edge_index[1].reshape(NW, NCHUNK, CHUNK)
    batch3 = batch.reshape(NBLK, 1, BLK)
    params = [(W1_0, B1_0, W2_0, B2_0, G_0, BT_0),
              (W1_1, B1_1, W2_1, B2_1, G_1, BT_1),
              (W1_2, B1_2, W2_2, B2_2, G_2, BT_2)]
    z = x
    pools = []
    sc_agg = _get_sc_agg()
    for (W1, B1, W2, B2, Gm, BT) in params:
        parts = sc_agg(z, src, dst)
        c, sums = _mlp_call(z, parts[0], parts[1], W1, B1, W2, B2)
        z, pool = _norm_call(c, sums, Gm, BT, batch3)
        pools.append(pool)
    return jnp.concatenate(pools, axis=1)


# speedup vs baseline: 10.2901x; 1.2847x over previous
import functools

import jax
import jax.numpy as jnp
from jax import lax
from jax.experimental import pallas as pl
from jax.experimental.pallas import tpu as pltpu
from jax.experimental.pallas import tpu_sc as plsc

N, D, E, G = 10000, 128, 320000, 128


NC, NS = 2, 16
NW = NC * NS
EPW = E // NW
CHUNK = 80
NCHUNK = EPW // CHUNK
NBUF = 4
NSLOT = 8
VCH = 128
WBT = 10
RPT = N // WBT
ZROWS = 40

def _sc_agg_body(z_hbm, src_hbm, dst_hbm, out_hbm, acc_sh, src_idx, dst_idx,
                 buf_0, buf_1, buf_2, buf_3,
                 gs_0, gs_1, gs_2, gs_3,
                 ss_0, ss_1, ss_2, ss_3,
                 is_0, is_1, is_2, is_3):
    cid = lax.axis_index("c")
    sid = lax.axis_index("s")
    wid = cid * NS + sid

    bufs = (buf_0, buf_1, buf_2, buf_3)
    gsems = (gs_0, gs_1, gs_2, gs_3)
    ssems = (ss_0, ss_1, ss_2, ss_3)
    isems = (is_0, is_1, is_2, is_3)

    def _zrow(i, carry):
        for j in range(D // 16):
            buf_0[i, pl.ds(j * 16, 16)] = jnp.zeros((16,), jnp.float32)
        return carry

    lax.fori_loop(0, ZROWS, _zrow, 0)

    @pl.when(sid < WBT)
    def _zero():
        for k in range(RPT // ZROWS):
            pltpu.sync_copy(buf_0.at[pl.ds(0, ZROWS)],
                            acc_sh.at[pl.ds(sid * RPT + k * ZROWS, ZROWS)])

    plsc.subcore_barrier()

    base = wid * EPW

    def fire_idx(v, sk):
        cw = lax.rem(v, NCHUNK) if not isinstance(v, int) else v % NCHUNK
        slot = lax.rem(v, NSLOT) if not isinstance(v, int) else v % NSLOT
        off = pl.multiple_of(base + cw * CHUNK, 16)
        soff = pl.multiple_of(slot * CHUNK, 16)
        pltpu.async_copy(src_hbm.at[pl.ds(off, CHUNK)],
                         src_idx.at[pl.ds(soff, CHUNK)], isems[sk])
        pltpu.async_copy(dst_hbm.at[pl.ds(off, CHUNK)],
                         dst_idx.at[slot], isems[sk])

    def wait_idx(sk):
        pltpu.make_async_copy(src_hbm.at[pl.ds(0, CHUNK)],
                              src_idx.at[pl.ds(0, CHUNK)], isems[sk]).wait()
        pltpu.make_async_copy(dst_hbm.at[pl.ds(0, CHUNK)],
                              dst_idx.at[0], isems[sk]).wait()

    def fire_gather(v, bk):
        slot = lax.rem(v, NSLOT) if not isinstance(v, int) else v % NSLOT
        soff = pl.multiple_of(slot * CHUNK, 16)
        pltpu.async_copy(z_hbm.at[src_idx.at[pl.ds(soff, CHUNK)]], bufs[bk],
                         gsems[bk])

    def wait_gather(bk):
        pltpu.make_async_copy(z_hbm.at[src_idx.at[pl.ds(0, CHUNK)]],
                              bufs[bk], gsems[bk]).wait()

    def fire_scatter(i, bk):
        slot = lax.rem(i, NSLOT) if not isinstance(i, int) else i % NSLOT
        pltpu.async_copy(bufs[bk], acc_sh.at[dst_idx.at[slot]], ssems[bk],
                         add=True)

    def drain_scatter(bk):
        pltpu.make_async_copy(bufs[bk], acc_sh.at[dst_idx.at[0]],
                              ssems[bk]).wait()

    for v in range(4):
        fire_idx(v, v)
    for v in range(2):
        wait_idx(v)
        fire_gather(v, v)

    def _step(j, carry):
        for k in range(NBUF):
            i = NBUF * j + k
            wait_gather(k)

            @pl.when(i < NCHUNK)
            def _(i=i, k=k):
                fire_scatter(i, k)

            @pl.when(jnp.logical_and(i >= 2, i <= NCHUNK + 1))
            def _(k=k):
                drain_scatter((k + 2) % NBUF)

            @pl.when(jnp.logical_and(i >= 0, i + 2 <= VCH - 1))
            def _(i=i, k=k):
                wait_idx((k + 2) % NBUF)
                fire_gather(i + 2, (k + 2) % NBUF)

            @pl.when(i + 4 <= VCH - 1)
            def _(i=i, k=k):
                fire_idx(i + 4, k)

        return carry

    lax.fori_loop(0, VCH // NBUF, _step, 0)
    plsc.subcore_barrier()

    @pl.when(sid < WBT)
    def _writeback():
        pltpu.sync_copy(acc_sh.at[pl.ds(sid * RPT, RPT)],
                        out_hbm.at[cid, pl.ds(sid * RPT, RPT)])


@functools.lru_cache(maxsize=None)
def _get_sc_agg():
    mesh = plsc.VectorSubcoreMesh(core_axis_name="c", subcore_axis_name="s")
    return pl.kernel(
        _sc_agg_body,
        out_type=jax.ShapeDtypeStruct((NC, N, D), jnp.float32),
        mesh=mesh,
        scratch_types=(
            [pltpu.VMEM_SHARED((N, D), jnp.float32)]
            + [pltpu.VMEM((NSLOT * CHUNK,), jnp.int32)]
            + [pltpu.VMEM((NSLOT, CHUNK), jnp.int32)]
            + [pltpu.VMEM((CHUNK, D), jnp.float32)] * NBUF
            + [pltpu.SemaphoreType.DMA] * 12
        ),
    )


BLK = 2000
NBLK = N // BLK


def _mlp_body(z_ref, p0_ref, p1_ref, w1_ref, b1_ref, w2_ref, b2_ref,
              c_ref, s_ref):
    h = z_ref[...] + p0_ref[...] + p1_ref[...]
    a = jnp.maximum(
        jnp.dot(h, w1_ref[...],
                preferred_element_type=jnp.float32) + b1_ref[...], 0.0)
    b = jnp.dot(a, w2_ref[...],
                preferred_element_type=jnp.float32) + b2_ref[...]
    c = jnp.maximum(b, 0.0)
    c_ref[...] = c

    bsum = jnp.sum(c, axis=0, keepdims=True)
    bmean = bsum * (1.0 / BLK)
    bm2 = jnp.sum((c - bmean) ** 2, axis=0, keepdims=True)
    s_ref[...] = jnp.concatenate(
        [bsum, bm2, jnp.zeros((6, D), jnp.float32)], axis=0)[None]


def _mlp_call(z, p0, p1, W1, B1, W2, B2):
    return pl.pallas_call(
        _mlp_body,
        grid=(NBLK,),
        in_specs=[
            pl.BlockSpec((BLK, D), lambda i: (i, 0)),
            pl.BlockSpec((BLK, D), lambda i: (i, 0)),
            pl.BlockSpec((BLK, D), lambda i: (i, 0)),
            pl.BlockSpec((D, D), lambda i: (0, 0)),
            pl.BlockSpec((1, D), lambda i: (0, 0)),
            pl.BlockSpec((D, D), lambda i: (0, 0)),
            pl.BlockSpec((1, D), lambda i: (0, 0)),
        ],
        out_specs=[
            pl.BlockSpec((BLK, D), lambda i: (i, 0)),
            pl.BlockSpec((1, 8, D), lambda i: (i, 0, 0)),
        ],
        out_shape=[
            jax.ShapeDtypeStruct((N, D), jnp.float32),
            jax.ShapeDtypeStruct((NBLK, 8, D), jnp.float32),
        ],
    )(z, p0, p1, W1, B1.reshape(1, D), W2, B2.reshape(1, D))


def _norm_body(c_ref, s_ref, g_ref, bt_ref, batch_ref, z_ref, pool_ref):
    s = s_ref[...]
    bsum = s[:, 0, :]
    bm2 = s[:, 1, :]
    mean = jnp.sum(bsum, axis=0, keepdims=True) * (1.0 / N)
    dev = bsum * (1.0 / BLK) - mean
    var = (jnp.sum(bm2, axis=0, keepdims=True)
           + BLK * jnp.sum(dev * dev, axis=0, keepdims=True)) * (1.0 / N)
    inv = lax.rsqrt(var + 1e-5) * g_ref[...]
    zb = (c_ref[...] - mean) * inv + bt_ref[...]
    z_ref[...] = zb

    brow = batch_ref[0, 0, :]
    onehot = (brow[None, :] ==
              lax.broadcasted_iota(jnp.int32, (G, BLK), 0)).astype(jnp.float32)
    contrib = jnp.dot(onehot, zb, preferred_element_type=jnp.float32,
                      precision=lax.Precision.HIGHEST)

    @pl.when(pl.program_id(0) == 0)
    def _():
        pool_ref[...] = jnp.zeros_like(pool_ref)

    pool_ref[...] += contrib


def _norm_call(c, sums, Gm, BT, batch3):
    return pl.pallas_call(
        _norm_body,
        grid=(NBLK,),
        in_specs=[
            pl.BlockSpec((BLK, D), lambda i: (i, 0)),
            pl.BlockSpec((NBLK, 8, D), lambda i: (0, 0, 0)),
            pl.BlockSpec((1, D), lambda i: (0, 0)),
            pl.BlockSpec((1, D), lambda i: (0, 0)),
            pl.BlockSpec((1, 1, BLK), lambda i: (i, 0, 0)),
        ],
        out_specs=[
            pl.BlockSpec((BLK, D), lambda i: (i, 0)),
            pl.BlockSpec((G, D), lambda i: (0, 0)),
        ],
        out_shape=[
            jax.ShapeDtypeStruct((N, D), jnp.float32),
            jax.ShapeDtypeStruct((G, D), jnp.float32),
        ],
    )(c, sums, Gm.reshape(1, D), BT.reshape(1, D), batch3)


def kernel(x, edge_index, batch, W1_0, B1_0, W2_0, B2_0, G_0, BT_0,
           W1_1, B1_1, W2_1, B2_1, G_1, BT_1,
           W1_2, B1_2, W2_2, B2_2, G_2, BT_2):
    src = edge_index[0]
    dst = edge_index[1]
    batch3 = batch.reshape(NBLK, 1, BLK)
    params = [(W1_0, B1_0, W2_0, B2_0, G_0, BT_0),
              (W1_1, B1_1, W2_1, B2_1, G_1, BT_1),
              (W1_2, B1_2, W2_2, B2_2, G_2, BT_2)]
    z = x
    pools = []
    sc_agg = _get_sc_agg()
    for (W1, B1, W2, B2, Gm, BT) in params:
        parts = sc_agg(z, src, dst)
        c, sums = _mlp_call(z, parts[0], parts[1], W1, B1, W2, B2)
        z, pool = _norm_call(c, sums, Gm, BT, batch3)
        pools.append(pool)
    return jnp.concatenate(pools, axis=1)


# speedup vs baseline: 10.2987x; 1.0008x over previous
import functools

import jax
import jax.numpy as jnp
from jax import lax
from jax.experimental import pallas as pl
from jax.experimental.pallas import tpu as pltpu
from jax.experimental.pallas import tpu_sc as plsc

N, D, E, G = 10000, 128, 320000, 128


NC, NS = 2, 16
NW = NC * NS
EPW = E // NW
CHUNK = 80
NCHUNK = EPW // CHUNK
NBUF = 4
NSLOT = 8
VCH = 128
WBT = 10
RPT = N // WBT
ZROWS = 40

def _sc_agg_body(z_hbm, src_hbm, dst_hbm, out_hbm, acc_sh, src_idx, dst_idx,
                 buf_0, buf_1, buf_2, buf_3,
                 gs_0, gs_1, gs_2, gs_3,
                 ss_0, ss_1, ss_2, ss_3,
                 is_0, is_1, is_2, is_3):
    cid = lax.axis_index("c")
    sid = lax.axis_index("s")
    wid = cid * NS + sid

    bufs = (buf_0, buf_1, buf_2, buf_3)
    gsems = (gs_0, gs_1, gs_2, gs_3)
    ssems = (ss_0, ss_1, ss_2, ss_3)
    isems = (is_0, is_1, is_2, is_3)

    def _zrow(i, carry):
        for j in range(D // 16):
            buf_0[i, pl.ds(j * 16, 16)] = jnp.zeros((16,), jnp.float32)
        return carry

    lax.fori_loop(0, ZROWS, _zrow, 0)

    @pl.when(sid < WBT)
    def _zero():
        for k in range(RPT // ZROWS):
            pltpu.sync_copy(buf_0.at[pl.ds(0, ZROWS)],
                            acc_sh.at[pl.ds(sid * RPT + k * ZROWS, ZROWS)])

    plsc.subcore_barrier()

    base = wid * EPW

    def fire_idx(v, sk):
        cw = lax.rem(v, NCHUNK) if not isinstance(v, int) else v % NCHUNK
        slot = lax.rem(v, NSLOT) if not isinstance(v, int) else v % NSLOT
        off = pl.multiple_of(base + cw * CHUNK, 16)
        soff = pl.multiple_of(slot * CHUNK, 16)
        pltpu.async_copy(src_hbm.at[pl.ds(off, CHUNK)],
                         src_idx.at[pl.ds(soff, CHUNK)], isems[sk])
        pltpu.async_copy(dst_hbm.at[pl.ds(off, CHUNK)],
                         dst_idx.at[slot], isems[sk])

    def wait_idx(sk):
        pltpu.make_async_copy(src_hbm.at[pl.ds(0, CHUNK)],
                              src_idx.at[pl.ds(0, CHUNK)], isems[sk]).wait()
        pltpu.make_async_copy(dst_hbm.at[pl.ds(0, CHUNK)],
                              dst_idx.at[0], isems[sk]).wait()

    HC = CHUNK // 2

    def fire_gather(v, bk):
        slot = lax.rem(v, NSLOT) if not isinstance(v, int) else v % NSLOT
        soff = pl.multiple_of(slot * CHUNK, 16)
        pltpu.async_copy(z_hbm.at[src_idx.at[pl.ds(soff, HC)]],
                         bufs[bk].at[pl.ds(0, HC)], gsems[bk])
        soff2 = pl.multiple_of(slot * CHUNK + HC, 8)
        pltpu.async_copy(z_hbm.at[src_idx.at[pl.ds(soff2, HC)]],
                         bufs[bk].at[pl.ds(HC, HC)], gsems[bk])

    def wait_gather(bk):
        for h in range(2):
            pltpu.make_async_copy(z_hbm.at[src_idx.at[pl.ds(0, HC)]],
                                  bufs[bk].at[pl.ds(h * HC, HC)],
                                  gsems[bk]).wait()

    def fire_scatter(i, bk):
        slot = lax.rem(i, NSLOT) if not isinstance(i, int) else i % NSLOT
        pltpu.async_copy(bufs[bk], acc_sh.at[dst_idx.at[slot]], ssems[bk],
                         add=True)

    def drain_scatter(bk):
        pltpu.make_async_copy(bufs[bk], acc_sh.at[dst_idx.at[0]],
                              ssems[bk]).wait()

    for v in range(4):
        fire_idx(v, v)
    for v in range(2):
        wait_idx(v)
        fire_gather(v, v)

    def _step(j, carry):
        for k in range(NBUF):
            i = NBUF * j + k
            wait_gather(k)

            @pl.when(i < NCHUNK)
            def _(i=i, k=k):
                fire_scatter(i, k)

            @pl.when(jnp.logical_and(i >= 2, i <= NCHUNK + 1))
            def _(k=k):
                drain_scatter((k + 2) % NBUF)

            @pl.when(jnp.logical_and(i >= 0, i + 2 <= VCH - 1))
            def _(i=i, k=k):
                wait_idx((k + 2) % NBUF)
                fire_gather(i + 2, (k + 2) % NBUF)

            @pl.when(i + 4 <= VCH - 1)
            def _(i=i, k=k):
                fire_idx(i + 4, k)

        return carry

    lax.fori_loop(0, VCH // NBUF, _step, 0)
    plsc.subcore_barrier()

    @pl.when(sid < WBT)
    def _writeback():
        pltpu.sync_copy(acc_sh.at[pl.ds(sid * RPT, RPT)],
                        out_hbm.at[cid, pl.ds(sid * RPT, RPT)])


@functools.lru_cache(maxsize=None)
def _get_sc_agg():
    mesh = plsc.VectorSubcoreMesh(core_axis_name="c", subcore_axis_name="s")
    return pl.kernel(
        _sc_agg_body,
        out_type=jax.ShapeDtypeStruct((NC, N, D), jnp.float32),
        mesh=mesh,
        scratch_types=(
            [pltpu.VMEM_SHARED((N, D), jnp.float32)]
            + [pltpu.VMEM((NSLOT * CHUNK,), jnp.int32)]
            + [pltpu.VMEM((NSLOT, CHUNK), jnp.int32)]
            + [pltpu.VMEM((CHUNK, D), jnp.float32)] * NBUF
            + [pltpu.SemaphoreType.DMA] * 12
        ),
    )


BLK = 2000
NBLK = N // BLK


def _mlp_body(z_ref, p0_ref, p1_ref, w1_ref, b1_ref, w2_ref, b2_ref,
              c_ref, s_ref):
    h = z_ref[...] + p0_ref[...] + p1_ref[...]
    a = jnp.maximum(
        jnp.dot(h, w1_ref[...],
                preferred_element_type=jnp.float32) + b1_ref[...], 0.0)
    b = jnp.dot(a, w2_ref[...],
                preferred_element_type=jnp.float32) + b2_ref[...]
    c = jnp.maximum(b, 0.0)
    c_ref[...] = c

    bsum = jnp.sum(c, axis=0, keepdims=True)
    bmean = bsum * (1.0 / BLK)
    bm2 = jnp.sum((c - bmean) ** 2, axis=0, keepdims=True)
    s_ref[...] = jnp.concatenate(
        [bsum, bm2, jnp.zeros((6, D), jnp.float32)], axis=0)[None]


def _mlp_call(z, p0, p1, W1, B1, W2, B2):
    return pl.pallas_call(
        _mlp_body,
        grid=(NBLK,),
        in_specs=[
            pl.BlockSpec((BLK, D), lambda i: (i, 0)),
            pl.BlockSpec((BLK, D), lambda i: (i, 0)),
            pl.BlockSpec((BLK, D), lambda i: (i, 0)),
            pl.BlockSpec((D, D), lambda i: (0, 0)),
            pl.BlockSpec((1, D), lambda i: (0, 0)),
            pl.BlockSpec((D, D), lambda i: (0, 0)),
            pl.BlockSpec((1, D), lambda i: (0, 0)),
        ],
        out_specs=[
            pl.BlockSpec((BLK, D), lambda i: (i, 0)),
            pl.BlockSpec((1, 8, D), lambda i: (i, 0, 0)),
        ],
        out_shape=[
            jax.ShapeDtypeStruct((N, D), jnp.float32),
            jax.ShapeDtypeStruct((NBLK, 8, D), jnp.float32),
        ],
    )(z, p0, p1, W1, B1.reshape(1, D), W2, B2.reshape(1, D))


def _norm_body(c_ref, s_ref, g_ref, bt_ref, batch_ref, z_ref, pool_ref):
    s = s_ref[...]
    bsum = s[:, 0, :]
    bm2 = s[:, 1, :]
    mean = jnp.sum(bsum, axis=0, keepdims=True) * (1.0 / N)
    dev = bsum * (1.0 / BLK) - mean
    var = (jnp.sum(bm2, axis=0, keepdims=True)
           + BLK * jnp.sum(dev * dev, axis=0, keepdims=True)) * (1.0 / N)
    inv = lax.rsqrt(var + 1e-5) * g_ref[...]
    zb = (c_ref[...] - mean) * inv + bt_ref[...]
    z_ref[...] = zb

    brow = batch_ref[0, 0, :]
    onehot = (brow[None, :] ==
              lax.broadcasted_iota(jnp.int32, (G, BLK), 0)).astype(jnp.float32)
    contrib = jnp.dot(onehot, zb, preferred_element_type=jnp.float32,
                      precision=lax.Precision.HIGHEST)

    @pl.when(pl.program_id(0) == 0)
    def _():
        pool_ref[...] = jnp.zeros_like(pool_ref)

    pool_ref[...] += contrib


def _norm_call(c, sums, Gm, BT, batch3):
    return pl.pallas_call(
        _norm_body,
        grid=(NBLK,),
        in_specs=[
            pl.BlockSpec((BLK, D), lambda i: (i, 0)),
            pl.BlockSpec((NBLK, 8, D), lambda i: (0, 0, 0)),
            pl.BlockSpec((1, D), lambda i: (0, 0)),
            pl.BlockSpec((1, D), lambda i: (0, 0)),
            pl.BlockSpec((1, 1, BLK), lambda i: (i, 0, 0)),
        ],
        out_specs=[
            pl.BlockSpec((BLK, D), lambda i: (i, 0)),
            pl.BlockSpec((G, D), lambda i: (0, 0)),
        ],
        out_shape=[
            jax.ShapeDtypeStruct((N, D), jnp.float32),
            jax.ShapeDtypeStruct((G, D), jnp.float32),
        ],
    )(c, sums, Gm.reshape(1, D), BT.reshape(1, D), batch3)


def kernel(x, edge_index, batch, W1_0, B1_0, W2_0, B2_0, G_0, BT_0,
           W1_1, B1_1, W2_1, B2_1, G_1, BT_1,
           W1_2, B1_2, W2_2, B2_2, G_2, BT_2):
    src = edge_index[0]
    dst = edge_index[1]
    batch3 = batch.reshape(NBLK, 1, BLK)
    params = [(W1_0, B1_0, W2_0, B2_0, G_0, BT_0),
              (W1_1, B1_1, W2_1, B2_1, G_1, BT_1),
              (W1_2, B1_2, W2_2, B2_2, G_2, BT_2)]
    z = x
    pools = []
    sc_agg = _get_sc_agg()
    for (W1, B1, W2, B2, Gm, BT) in params:
        parts = sc_agg(z, src, dst)
        c, sums = _mlp_call(z, parts[0], parts[1], W1, B1, W2, B2)
        z, pool = _norm_call(c, sums, Gm, BT, batch3)
        pools.append(pool)
    return jnp.concatenate(pools, axis=1)
